# baseline (speedup 1.0000x reference)
import functools

import jax
import jax.numpy as jnp
from jax import lax
from jax.experimental import pallas as pl
from jax.experimental.pallas import tpu as pltpu
from jax.experimental.pallas import tpu_sc as plsc

VOCAB = 1000000
EMBED = 32
BATCH = 4096
SEQ = 200

NC = 2
NS = 16
NW = NC * NS
CW = 128
NCH = SEQ
NSLOT = 8
WSLOT = 4
ET = EMBED // 8


def _body(table_hbm, idx_hbm, out_hbm, idx_v, rows_v, trows_v, gsem, wsem):
    wid = lax.axis_index("s") * NC + lax.axis_index("c")
    pltpu.sync_copy(idx_hbm.at[wid], idx_v)

    iota16 = lax.iota(jnp.int32, 16)
    rot = [(iota16 + d) % 16 for d in range(16)]
    didx = [iota16 * CW + r for r in rot]

    def start_gather(j, slot):
        pltpu.async_copy(
            table_hbm.at[idx_v.at[j]],
            rows_v.at[pl.ds(slot * CW, CW), :],
            gsem.at[slot],
        )

    def wait_gather(j, slot):
        pltpu.make_async_copy(
            table_hbm.at[idx_v.at[j]],
            rows_v.at[pl.ds(slot * CW, CW), :],
            gsem.at[slot],
        ).wait()

    def start_wb(j, slot):
        for et in range(ET):
            pltpu.async_copy(
                trows_v.at[slot, pl.ds(et * 1024, 1024)],
                out_hbm.at[j, et, wid],
                wsem.at[slot],
            )

    def wait_wb(j, slot):
        for et in range(ET):
            pltpu.make_async_copy(
                trows_v.at[slot, pl.ds(et * 1024, 1024)],
                out_hbm.at[j, et, wid],
                wsem.at[slot],
            ).wait()

    def transpose(slot, wslot):
        for h in range(2):
            for g in range(CW // 16):
                src = rows_v.at[pl.ds(slot * CW + 16 * g, 16), :]
                col = iota16 + 16 * h
                dst = trows_v.at[wslot, pl.ds(2048 * h + 16 * g, 1936)]
                for d0 in range(0, 16, 8):
                    vs = [plsc.load_gather(src, [rot[d0 + k], col])
                          for k in range(8)]
                    for k in range(8):
                        plsc.store_scatter(dst, [didx[d0 + k]], vs[k])

    for b in range(NSLOT):
        start_gather(b, b)

    def group(i, carry):
        for b in range(NSLOT):
            j = NSLOT * i + b
            wb = b % WSLOT
            wait_gather(j, b)

            @pl.when(j >= WSLOT)
            def _():
                wait_wb(j - WSLOT, wb)

            transpose(b, wb)
            start_wb(j, wb)

            @pl.when(j + NSLOT < NCH)
            def _():
                start_gather(j + NSLOT, b)

        return carry

    lax.fori_loop(0, NCH // NSLOT, group, 0)

    for b in range(WSLOT):
        wait_wb(NCH - WSLOT + b, (NCH - WSLOT + b) % WSLOT)


NT = (VOCAB + CW - 1) // CW
TPW = 245
GS1 = 4


def _tbody(tt_hbm, out_hbm, rows_v, trows_v, gsem, wsem):
    wid = lax.axis_index("s") * NC + lax.axis_index("c")

    iota16 = lax.iota(jnp.int32, 16)
    rot = [(iota16 + d) % 16 for d in range(16)]
    ddx = [iota16 * EMBED + r for r in rot]

    def cond(t):
        return jnp.logical_and(t < TPW, wid * TPW + t < NT)

    def in_copies(t, slot):
        vt = wid * TPW + t
        return [
            pltpu.make_async_copy(
                tt_hbm.at[pl.ds(8 * et, 8), pl.ds(vt * CW, CW)],
                rows_v.at[pl.ds(slot * 32 + 8 * et, 8), :],
                gsem.at[slot],
            )
            for et in range(ET)
        ]

    def out_copy(t, slot, n):
        vt = wid * TPW + t
        return pltpu.make_async_copy(
            trows_v.at[pl.ds(slot * 4096, n)],
            out_hbm.at[pl.ds(vt * 4096, n)],
            wsem.at[slot],
        )

    def start_in(t, slot):
        @pl.when(cond(t))
        def _():
            for c in in_copies(t, slot):
                c.start()

    def wait_in(t, slot):
        @pl.when(cond(t))
        def _():
            for c in in_copies(t, slot):
                c.wait()

    def start_out(t, slot):
        vt = wid * TPW + t

        @pl.when(jnp.logical_and(cond(t), vt != NT - 1))
        def _():
            out_copy(t, slot, 4096).start()

        @pl.when(jnp.logical_and(cond(t), vt == NT - 1))
        def _():
            out_copy(t, slot, 2048).start()

    def wait_out(t, slot):
        vt = wid * TPW + t

        @pl.when(jnp.logical_and(cond(t), vt != NT - 1))
        def _():
            out_copy(t, slot, 4096).wait()

        @pl.when(jnp.logical_and(cond(t), vt == NT - 1))
        def _():
            out_copy(t, slot, 2048).wait()

    def transpose(slot, wslot):
        for rh in range(2):
            src = rows_v.at[pl.ds(slot * 32 + 16 * rh, 16), :]
            for q in range(8):
                col = iota16 + 16 * q
                dst = trows_v.at[pl.ds(wslot * 4096 + 512 * q + 16 * rh, 496)]
                for d0 in (0, 8):
                    vs = [plsc.load_gather(src, [rot[d0 + k], col])
                          for k in range(8)]
                    for k in range(8):
                        plsc.store_scatter(dst, [ddx[d0 + k]], vs[k])

    for b in range(GS1):
        start_in(b, b)

    def group(i, carry):
        for b in range(GS1):
            t = GS1 * i + b
            wait_in(t, b)

            @pl.when(i >= 1)
            def _():
                wait_out(t - GS1, b)

            transpose(b, b)
            start_out(t, b)
            start_in(t + GS1, b)
        return carry

    lax.fori_loop(0, 62, group, 0)

    for t in range(248 - GS1, TPW):
        wait_out(t, t % GS1)


_transpose_call = functools.partial(
    pl.kernel,
    out_type=jax.ShapeDtypeStruct((VOCAB * EMBED,), jnp.float32),
    mesh=plsc.VectorSubcoreMesh(core_axis_name="c", subcore_axis_name="s"),
    scratch_types=[
        pltpu.VMEM((GS1 * 32, CW), jnp.float32),
        pltpu.VMEM((GS1 * 4096,), jnp.float32),
        pltpu.SemaphoreType.DMA((GS1,)),
        pltpu.SemaphoreType.DMA((GS1,)),
    ],
    compiler_params=pltpu.CompilerParams(
        use_tc_tiling_on_sc=True, needs_layout_passes=False
    ),
)(_tbody)


_gather_call = functools.partial(
    pl.kernel,
    out_type=jax.ShapeDtypeStruct((SEQ, ET, NW, 1024), jnp.float32),
    mesh=plsc.VectorSubcoreMesh(core_axis_name="c", subcore_axis_name="s"),
    scratch_types=[
        pltpu.VMEM((NCH, CW), jnp.int32),
        pltpu.VMEM((NSLOT * CW, EMBED), jnp.float32),
        pltpu.VMEM((WSLOT, ET * 8 * CW), jnp.float32),
        pltpu.SemaphoreType.DMA((NSLOT,)),
        pltpu.SemaphoreType.DMA((WSLOT,)),
    ],
    compiler_params=pltpu.CompilerParams(
        use_tc_tiling_on_sc=False, needs_layout_passes=False
    ),
)(_body)


@jax.jit
def kernel(indices, table):
    idx3d = jnp.transpose(jnp.reshape(indices, (NW, CW, SEQ)), (0, 2, 1))
    tlin = _transpose_call(jnp.swapaxes(table, 0, 1))
    out = _gather_call(jnp.reshape(tlin, (VOCAB, EMBED)), idx3d)
    out5 = jnp.reshape(out, (SEQ, ET, NW, 8, CW))
    return jnp.reshape(jnp.transpose(out5, (2, 4, 0, 1, 3)), (BATCH, SEQ, EMBED))


# speedup vs baseline: 1.1116x; 1.1116x over previous
import functools

import jax
import jax.numpy as jnp
from jax import lax
from jax.experimental import pallas as pl
from jax.experimental.pallas import tpu as pltpu
from jax.experimental.pallas import tpu_sc as plsc

VOCAB = 1000000
EMBED = 32
BATCH = 4096
SEQ = 200

NC = 2
NS = 16
NW = NC * NS
CW = 128
NCH = SEQ
NSLOT = 8
WSLOT = 4
ET = EMBED // 8


def _body(table_hbm, idx_hbm, out_hbm, idx_v, rows_v, trows_v, gsem, wsem):
    wid = lax.axis_index("s") * NC + lax.axis_index("c")
    pltpu.sync_copy(idx_hbm.at[wid], idx_v)

    iota16 = lax.iota(jnp.int32, 16)
    rot = [(iota16 + d) % 16 for d in range(16)]
    didx = [iota16 * CW + r for r in rot]

    def start_gather(j, slot):
        pltpu.async_copy(
            table_hbm.at[idx_v.at[j]],
            rows_v.at[pl.ds(slot * CW, CW), :],
            gsem.at[slot],
        )

    def wait_gather(j, slot):
        pltpu.make_async_copy(
            table_hbm.at[idx_v.at[j]],
            rows_v.at[pl.ds(slot * CW, CW), :],
            gsem.at[slot],
        ).wait()

    def start_wb(j, slot):
        for et in range(ET):
            pltpu.async_copy(
                trows_v.at[slot, pl.ds(et * 1024, 1024)],
                out_hbm.at[j, et, wid],
                wsem.at[slot],
            )

    def wait_wb(j, slot):
        for et in range(ET):
            pltpu.make_async_copy(
                trows_v.at[slot, pl.ds(et * 1024, 1024)],
                out_hbm.at[j, et, wid],
                wsem.at[slot],
            ).wait()

    def transpose(slot, wslot):
        for h in range(2):
            for g in range(CW // 16):
                src = rows_v.at[pl.ds(slot * CW + 16 * g, 16), :]
                col = iota16 + 16 * h
                dst = trows_v.at[wslot, pl.ds(2048 * h + 16 * g, 1936)]
                for d0 in range(0, 16, 8):
                    vs = [plsc.load_gather(src, [rot[d0 + k], col])
                          for k in range(8)]
                    for k in range(8):
                        plsc.store_scatter(dst, [didx[d0 + k]], vs[k])

    for b in range(NSLOT):
        start_gather(b, b)

    def group(i, carry):
        for b in range(NSLOT):
            j = NSLOT * i + b
            wb = b % WSLOT
            wait_gather(j, b)

            @pl.when(j >= WSLOT)
            def _():
                wait_wb(j - WSLOT, wb)

            transpose(b, wb)
            start_wb(j, wb)

            @pl.when(j + NSLOT < NCH)
            def _():
                start_gather(j + NSLOT, b)

        return carry

    lax.fori_loop(0, NCH // NSLOT, group, 0)

    for b in range(WSLOT):
        wait_wb(NCH - WSLOT + b, (NCH - WSLOT + b) % WSLOT)


NIT = (VOCAB + 255) // 256
TPW = 123
GS1 = 2


def _tbody(tt_hbm, out_hbm, rows_v, trows_v, gsem, wsem):
    wid = lax.axis_index("s") * NC + lax.axis_index("c")

    iota16 = lax.iota(jnp.int32, 16)
    rot = [(iota16 + d) % 16 for d in range(16)]
    ddx = [iota16 * EMBED + r for r in rot]

    def cond(t):
        return jnp.logical_and(t < TPW, wid * TPW + t < NIT)

    def in_copies(t, slot, n):
        it = wid * TPW + t
        return [
            pltpu.make_async_copy(
                tt_hbm.at[pl.ds(8 * et, 8), pl.ds(it * 256, n)],
                rows_v.at[pl.ds(slot * 32 + 8 * et, 8), pl.ds(0, n)],
                gsem.at[slot],
            )
            for et in range(ET)
        ]

    def out_copy(t, slot, n):
        it = wid * TPW + t
        return pltpu.make_async_copy(
            trows_v.at[pl.ds(slot * 8192, n)],
            out_hbm.at[pl.ds(it * 8192, n)],
            wsem.at[slot],
        )

    def _variants(t, slot, make, ns):
        it = wid * TPW + t

        @pl.when(jnp.logical_and(cond(t), it != NIT - 1))
        def _():
            make(t, slot, ns[0])

        @pl.when(jnp.logical_and(cond(t), it == NIT - 1))
        def _():
            make(t, slot, ns[1])

    def start_in(t, slot):
        _variants(t, slot,
                  lambda t, s, n: [c.start() for c in in_copies(t, s, n)],
                  (256, 128))

    def wait_in(t, slot):
        _variants(t, slot,
                  lambda t, s, n: [c.wait() for c in in_copies(t, s, n)],
                  (256, 128))

    def start_out(t, slot):
        _variants(t, slot, lambda t, s, n: out_copy(t, s, n).start(),
                  (8192, 2048))

    def wait_out(t, slot):
        _variants(t, slot, lambda t, s, n: out_copy(t, s, n).wait(),
                  (8192, 2048))

    def transpose(slot, wslot):
        for rh in range(2):
            src = rows_v.at[pl.ds(slot * 32 + 16 * rh, 16), :]
            for vh in range(2):
                for q in range(8):
                    col = iota16 + 128 * vh + 16 * q
                    dst = trows_v.at[
                        pl.ds(wslot * 8192 + 4096 * vh + 512 * q + 16 * rh,
                              496)]
                    for d0 in (0, 8):
                        vs = [plsc.load_gather(src, [rot[d0 + k], col])
                              for k in range(8)]
                        for k in range(8):
                            plsc.store_scatter(dst, [ddx[d0 + k]], vs[k])

    for b in range(GS1):
        start_in(b, b)

    def group(i, carry):
        for b in range(GS1):
            t = GS1 * i + b
            wait_in(t, b)

            @pl.when(i >= 1)
            def _():
                wait_out(t - GS1, b)

            transpose(b, b)
            start_out(t, b)
            start_in(t + GS1, b)
        return carry

    lax.fori_loop(0, 62, group, 0)

    for t in range(124 - GS1, TPW):
        wait_out(t, t % GS1)


_transpose_call = functools.partial(
    pl.kernel,
    out_type=jax.ShapeDtypeStruct((VOCAB * EMBED,), jnp.float32),
    mesh=plsc.VectorSubcoreMesh(core_axis_name="c", subcore_axis_name="s"),
    scratch_types=[
        pltpu.VMEM((GS1 * 32, 256), jnp.float32),
        pltpu.VMEM((GS1 * 8192,), jnp.float32),
        pltpu.SemaphoreType.DMA((GS1,)),
        pltpu.SemaphoreType.DMA((GS1,)),
    ],
    compiler_params=pltpu.CompilerParams(
        use_tc_tiling_on_sc=True, needs_layout_passes=False
    ),
)(_tbody)


_gather_call = functools.partial(
    pl.kernel,
    out_type=jax.ShapeDtypeStruct((SEQ, ET, NW, 1024), jnp.float32),
    mesh=plsc.VectorSubcoreMesh(core_axis_name="c", subcore_axis_name="s"),
    scratch_types=[
        pltpu.VMEM((NCH, CW), jnp.int32),
        pltpu.VMEM((NSLOT * CW, EMBED), jnp.float32),
        pltpu.VMEM((WSLOT, ET * 8 * CW), jnp.float32),
        pltpu.SemaphoreType.DMA((NSLOT,)),
        pltpu.SemaphoreType.DMA((WSLOT,)),
    ],
    compiler_params=pltpu.CompilerParams(
        use_tc_tiling_on_sc=False, needs_layout_passes=False
    ),
)(_body)


@jax.jit
def kernel(indices, table):
    idx3d = jnp.transpose(jnp.reshape(indices, (NW, CW, SEQ)), (0, 2, 1))
    tlin = _transpose_call(jnp.swapaxes(table, 0, 1))
    out = _gather_call(jnp.reshape(tlin, (VOCAB, EMBED)), idx3d)
    out5 = jnp.reshape(out, (SEQ, ET, NW, 8, CW))
    return jnp.reshape(jnp.transpose(out5, (2, 4, 0, 1, 3)), (BATCH, SEQ, EMBED))
